# Initial kernel scaffold; baseline (speedup 1.0000x reference)
import jax
import jax.numpy as jnp
from jax.experimental import pallas as pl


def kernel(pix_to_face, bary_coords, verts, faces):
    raise NotImplementedError("write your pallas kernel here")


# speedup vs baseline: 7.7467x; 7.7467x over previous
import functools

import jax
import jax.numpy as jnp
from jax.experimental import pallas as pl
from jax.experimental.pallas import tpu as pltpu
from jax.experimental.pallas import tpu_sc as plsc

_LANES = 16
_WINDOW = 128


def _sc_gather_rows(table, idx):
    n, d = idx.shape[0], table.shape[1]
    nwin = n // _WINDOW
    mesh = plsc.VectorSubcoreMesh(core_axis_name="c", subcore_axis_name="s")

    @functools.partial(
        pl.kernel,
        out_type=jax.ShapeDtypeStruct((n, d), table.dtype),
        mesh=mesh,
        compiler_params=pltpu.CompilerParams(use_tc_tiling_on_sc=False),
    )
    def gather_kernel(table_hbm, idx_hbm, out_hbm):
        def body(idx_vmem, out_vmem):
            pltpu.sync_copy(table_hbm.at[idx_vmem.at[0]], out_vmem)

        pltpu.emit_pipeline(
            body,
            grid=(nwin,),
            in_specs=[pl.BlockSpec((1, _WINDOW), lambda i: (0, i))],
            out_specs=[pl.BlockSpec((_WINDOW, d), lambda i: (i, 0))],
            core_axis_name=("c", "s"),
            dimension_semantics=(pltpu.PARALLEL,),
        )(idx_hbm, out_hbm)

    return gather_kernel(table, idx.reshape(1, n))


def _dist_body(b0, b1, b2, x00, x01, x02, x10, x11, x12, x20, x21, x22, o):
    p0 = b0[...] * x00[...] + b1[...] * x10[...] + b2[...] * x20[...]
    p1 = b0[...] * x01[...] + b1[...] * x11[...] + b2[...] * x21[...]
    p2 = b0[...] * x02[...] + b1[...] * x12[...] + b2[...] * x22[...]
    o[...] = jnp.sqrt(p0 * p0 + p1 * p1 + p2 * p2)


def _dist(args, total):
    cols = 512
    rows = total // cols
    br = 256
    spec = pl.BlockSpec((br, cols), lambda i: (i, 0))
    out = pl.pallas_call(
        _dist_body,
        grid=(rows // br,),
        in_specs=[spec] * 12,
        out_specs=spec,
        out_shape=jax.ShapeDtypeStruct((rows, cols), jnp.float32),
    )(*[a.reshape(rows, cols) for a in args])
    return out.reshape(total)


def kernel(pix_to_face, bary_coords, verts, faces):
    n, h, w, k = pix_to_face.shape
    f = faces.shape[0]
    b = n * h * w * k

    verts_pad = jnp.pad(verts.astype(jnp.float32), ((0, 0), (0, _LANES - 3)))
    faces32 = faces.astype(jnp.int32)
    fp = ((f + _WINDOW - 1) // _WINDOW) * _WINDOW
    faces_pad = jnp.pad(faces32, ((0, fp - f), (0, 0)))
    corner_idx = faces_pad.T.reshape(-1)
    corner_rows = _sc_gather_rows(verts_pad, corner_idx)
    tbl = jnp.concatenate(
        [corner_rows[0 * fp:0 * fp + f, 0:3],
         corner_rows[1 * fp:1 * fp + f, 0:3],
         corner_rows[2 * fp:2 * fp + f, 0:3],
         jnp.zeros((f, _LANES - 9), jnp.float32)], axis=1)

    idx = pix_to_face.astype(jnp.int32).transpose(3, 0, 1, 2).reshape(-1)
    g = _sc_gather_rows(tbl, idx)

    bary_t = bary_coords.astype(jnp.float32).transpose(4, 3, 0, 1, 2).reshape(3, b)
    args = [bary_t[0], bary_t[1], bary_t[2]] + [
        g[:, 3 * j + c] for j in range(3) for c in range(3)]
    d = _dist(args, b).reshape(k, n, h, w)
    return tuple(d[i].reshape(n, h, w, 1) for i in range(k))


# speedup vs baseline: 23.3254x; 3.0110x over previous
import functools

import jax
import jax.numpy as jnp
from jax.experimental import pallas as pl
from jax.experimental.pallas import tpu as pltpu
from jax.experimental.pallas import tpu_sc as plsc

_LANES = 16
_WINDOW = 128


def _sc_gather_rows(table, idx):
    n, d = idx.shape[0], table.shape[1]
    nwin = n // _WINDOW
    mesh = plsc.VectorSubcoreMesh(core_axis_name="c", subcore_axis_name="s")

    @functools.partial(
        pl.kernel,
        out_type=jax.ShapeDtypeStruct((n, d), table.dtype),
        mesh=mesh,
        compiler_params=pltpu.CompilerParams(use_tc_tiling_on_sc=False),
    )
    def gather_kernel(table_hbm, idx_hbm, out_hbm):
        def body(idx_vmem, out_vmem):
            pltpu.sync_copy(table_hbm.at[idx_vmem.at[0]], out_vmem)

        pltpu.emit_pipeline(
            body,
            grid=(nwin,),
            in_specs=[pl.BlockSpec((1, _WINDOW), lambda i: (0, i))],
            out_specs=[pl.BlockSpec((_WINDOW, d), lambda i: (i, 0))],
            core_axis_name=("c", "s"),
            dimension_semantics=(pltpu.PARALLEL,),
        )(idx_hbm, out_hbm)

    return gather_kernel(table, idx.reshape(1, n))


def _dist_body(gt_ref, bt_ref, o_ref):
    b0, b1, b2 = bt_ref[0], bt_ref[1], bt_ref[2]
    p0 = b0 * gt_ref[0] + b1 * gt_ref[3] + b2 * gt_ref[6]
    p1 = b0 * gt_ref[1] + b1 * gt_ref[4] + b2 * gt_ref[7]
    p2 = b0 * gt_ref[2] + b1 * gt_ref[5] + b2 * gt_ref[8]
    o_ref[...] = jnp.sqrt(p0 * p0 + p1 * p1 + p2 * p2)


def _dist(gt, bt, total):
    cols = 1024
    rows = total // cols
    br = 128
    out = pl.pallas_call(
        _dist_body,
        grid=(rows // br,),
        in_specs=[
            pl.BlockSpec((9, br, cols), lambda i: (0, i, 0)),
            pl.BlockSpec((3, br, cols), lambda i: (0, i, 0)),
        ],
        out_specs=pl.BlockSpec((br, cols), lambda i: (i, 0)),
        out_shape=jax.ShapeDtypeStruct((rows, cols), jnp.float32),
    )(gt.reshape(gt.shape[0], rows, cols), bt.reshape(3, rows, cols))
    return out.reshape(total)


def kernel(pix_to_face, bary_coords, verts, faces):
    n, h, w, k = pix_to_face.shape
    f = faces.shape[0]
    b = n * h * w * k

    verts_pad = jnp.pad(verts.astype(jnp.float32), ((0, 0), (0, _LANES - 3)))
    faces32 = faces.astype(jnp.int32)
    fp = ((f + _WINDOW - 1) // _WINDOW) * _WINDOW
    faces_pad = jnp.pad(faces32, ((0, fp - f), (0, 0)))
    corner_idx = faces_pad.T.reshape(-1)
    corner_rows = _sc_gather_rows(verts_pad, corner_idx)
    tbl = jnp.concatenate(
        [corner_rows[0 * fp:0 * fp + f, 0:3],
         corner_rows[1 * fp:1 * fp + f, 0:3],
         corner_rows[2 * fp:2 * fp + f, 0:3],
         jnp.zeros((f, _LANES - 9), jnp.float32)], axis=1)

    idx = pix_to_face.astype(jnp.int32).transpose(3, 0, 1, 2).reshape(-1)
    g = _sc_gather_rows(tbl, idx)

    gt = g.T[:9]
    bt = bary_coords.astype(jnp.float32).transpose(4, 3, 0, 1, 2).reshape(3, b)
    d = _dist(gt, bt, b).reshape(k, n, h, w)
    return tuple(d[i].reshape(n, h, w, 1) for i in range(k))
